# baseline (speedup 1.0000x reference)
import functools

import jax
import jax.numpy as jnp
from jax import lax
from jax.experimental import pallas as pl
from jax.experimental.pallas import tpu as pltpu
from jax.experimental.pallas import tpu_sc as plsc

_NC = 2
_NS = 16
_NW = _NC * _NS
_LANES = 16


def _make_sc_sum(B, L, V, E):
    bpw = B // _NW
    assert B % _NW == 0 and E % _LANES == 0 and bpw % 128 == 0

    mesh = plsc.VectorSubcoreMesh(core_axis_name="c", subcore_axis_name="s")

    @functools.partial(
        pl.kernel,
        out_type=jax.ShapeDtypeStruct((B, E), jnp.float32),
        mesh=mesh,
        scratch_types=[
            pltpu.VMEM((L, bpw), jnp.int32),
            pltpu.VMEM((bpw, E), jnp.float32),
            pltpu.VMEM((bpw, E), jnp.float32),
            pltpu.SemaphoreType.DMA,
            pltpu.SemaphoreType.DMA,
        ],
    )
    def sc_sum(xT_hbm, embed_hbm, sum_hbm, idx_v, acc_v, acc2_v, sem, sem2):
        wid = lax.axis_index("s") * _NC + lax.axis_index("c")
        base = wid * bpw

        head = 8
        pltpu.sync_copy(
            xT_hbm.at[pl.ds(0, head), pl.ds(base, bpw)], idx_v.at[pl.ds(0, head)]
        )
        d0 = pltpu.async_copy(embed_hbm.at[idx_v.at[0]], acc_v, sem)
        d1 = pltpu.async_copy(embed_hbm.at[idx_v.at[1]], acc2_v, sem2)
        pltpu.sync_copy(
            xT_hbm.at[pl.ds(head, L - head), pl.ds(base, bpw)],
            idx_v.at[pl.ds(head, L - head)],
        )
        d0.wait()
        d1.wait()
        descs = [
            pltpu.async_copy(
                embed_hbm.at[idx_v.at[jj]],
                acc_v if jj % 2 == 0 else acc2_v,
                sem if jj % 2 == 0 else sem2,
                add=True,
            )
            for jj in range(2, L)
        ]
        for d in descs:
            d.wait()

        def mbody(r, carry):
            for c in range(0, E, _LANES):
                acc_v[r, pl.ds(c, _LANES)] = (
                    acc_v[r, pl.ds(c, _LANES)] + acc2_v[r, pl.ds(c, _LANES)]
                )
            return carry

        lax.fori_loop(0, bpw, mbody, 0)

        pltpu.sync_copy(acc_v, sum_hbm.at[pl.ds(base, bpw), :])

    return sc_sum


def _make_mlp(B, L, E, H, N):
    BK = 4096

    def body(sum_ref, x_ref, w1_ref, b1_ref, w2_ref, b2_ref, out_ref):
        s = sum_ref[...]
        cnt = jnp.sum(
            (x_ref[...] != 0).astype(jnp.float32), axis=1, keepdims=True
        )
        avg = s * (1.0 / jnp.maximum(cnt, 1e-6))
        h = lax.dot_general(
            avg, w1_ref[...], (((1,), (0,)), ((), ())),
            preferred_element_type=jnp.float32,
        ) + b1_ref[...]
        h = jnp.maximum(h, 0.0)
        out_ref[...] = lax.dot_general(
            h, w2_ref[...], (((1,), (0,)), ((), ())),
            preferred_element_type=jnp.float32,
        ) + b2_ref[...]

    return pl.pallas_call(
        body,
        grid=(B // BK,),
        in_specs=[
            pl.BlockSpec((BK, E), lambda i: (i, 0)),
            pl.BlockSpec((BK, L), lambda i: (i, 0)),
            pl.BlockSpec((E, H), lambda i: (0, 0)),
            pl.BlockSpec((1, H), lambda i: (0, 0)),
            pl.BlockSpec((H, N), lambda i: (0, 0)),
            pl.BlockSpec((1, N), lambda i: (0, 0)),
        ],
        out_specs=pl.BlockSpec((BK, N), lambda i: (i, 0)),
        out_shape=jax.ShapeDtypeStruct((B, N), jnp.float32),
    )


def kernel(x, embed, W1, b1, W2, b2):
    B, L = x.shape
    V, E = embed.shape
    H = W1.shape[1]
    N = W2.shape[1]

    xi = x.astype(jnp.int32)
    xT = jnp.transpose(xi)
    summed = _make_sc_sum(B, L, V, E)(xT, embed)

    return _make_mlp(B, L, E, H, N)(
        summed, xi, W1, b1.reshape(1, H), W2, b2.reshape(1, N)
    )


# speedup vs baseline: 1.0080x; 1.0080x over previous
import functools

import jax
import jax.numpy as jnp
from jax import lax
from jax.experimental import pallas as pl
from jax.experimental.pallas import tpu as pltpu
from jax.experimental.pallas import tpu_sc as plsc

_NC = 2
_NS = 16
_NW = _NC * _NS
_LANES = 16


def _make_sc_sum(B, L, V, E):
    bpw = B // _NW
    assert B % _NW == 0 and E % _LANES == 0 and bpw % 128 == 0

    mesh = plsc.VectorSubcoreMesh(core_axis_name="c", subcore_axis_name="s")

    @functools.partial(
        pl.kernel,
        out_type=jax.ShapeDtypeStruct((B, E), jnp.float32),
        mesh=mesh,
        scratch_types=[
            pltpu.VMEM((L, bpw), jnp.int32),
            pltpu.VMEM((bpw, E), jnp.float32),
            pltpu.VMEM((bpw, E), jnp.float32),
            pltpu.SemaphoreType.DMA,
            pltpu.SemaphoreType.DMA,
        ],
    )
    def sc_sum(xT_hbm, embed_hbm, sum_hbm, idx_v, acc_v, acc2_v, sem, sem2):
        wid = lax.axis_index("s") * _NC + lax.axis_index("c")
        base = wid * bpw

        head = 8
        pltpu.sync_copy(
            xT_hbm.at[pl.ds(0, head), pl.ds(base, bpw)], idx_v.at[pl.ds(0, head)]
        )
        d0 = pltpu.async_copy(embed_hbm.at[idx_v.at[0]], acc_v, sem)
        d1 = pltpu.async_copy(embed_hbm.at[idx_v.at[1]], acc2_v, sem2)
        pltpu.sync_copy(
            xT_hbm.at[pl.ds(head, L - head), pl.ds(base, bpw)],
            idx_v.at[pl.ds(head, L - head)],
        )
        d0.wait()
        d1.wait()
        descs = [
            pltpu.async_copy(
                embed_hbm.at[idx_v.at[jj]],
                acc_v if jj % 2 == 0 else acc2_v,
                sem if jj % 2 == 0 else sem2,
                add=True,
            )
            for jj in range(2, L)
        ]
        for d in descs:
            d.wait()

        def mbody(r, carry):
            for c in range(0, E, _LANES):
                acc_v[r, pl.ds(c, _LANES)] = (
                    acc_v[r, pl.ds(c, _LANES)] + acc2_v[r, pl.ds(c, _LANES)]
                )
            return carry

        lax.fori_loop(0, bpw, mbody, 0)

        pltpu.sync_copy(acc_v, sum_hbm.at[pl.ds(base, bpw), :])

    return sc_sum


def _make_mlp(B, L, E, H, N):
    BK = 2048

    def body(sum_ref, x_ref, w1_ref, b1_ref, w2_ref, b2_ref, out_ref):
        s = sum_ref[...]
        cnt = jnp.sum(
            (x_ref[...] != 0).astype(jnp.float32), axis=1, keepdims=True
        )
        avg = s * (1.0 / jnp.maximum(cnt, 1e-6))
        h = lax.dot_general(
            avg, w1_ref[...], (((1,), (0,)), ((), ())),
            preferred_element_type=jnp.float32,
        ) + b1_ref[...]
        h = jnp.maximum(h, 0.0)
        out_ref[...] = lax.dot_general(
            h, w2_ref[...], (((1,), (0,)), ((), ())),
            preferred_element_type=jnp.float32,
        ) + b2_ref[...]

    return pl.pallas_call(
        body,
        grid=(B // BK,),
        in_specs=[
            pl.BlockSpec((BK, E), lambda i: (i, 0)),
            pl.BlockSpec((BK, L), lambda i: (i, 0)),
            pl.BlockSpec((E, H), lambda i: (0, 0)),
            pl.BlockSpec((1, H), lambda i: (0, 0)),
            pl.BlockSpec((H, N), lambda i: (0, 0)),
            pl.BlockSpec((1, N), lambda i: (0, 0)),
        ],
        out_specs=pl.BlockSpec((BK, N), lambda i: (i, 0)),
        out_shape=jax.ShapeDtypeStruct((B, N), jnp.float32),
    )


def kernel(x, embed, W1, b1, W2, b2):
    B, L = x.shape
    V, E = embed.shape
    H = W1.shape[1]
    N = W2.shape[1]

    xi = x.astype(jnp.int32)
    xT = jnp.transpose(xi)
    summed = _make_sc_sum(B, L, V, E)(xT, embed)

    return _make_mlp(B, L, E, H, N)(
        summed, xi, W1, b1.reshape(1, H), W2, b2.reshape(1, N)
    )
